# Initial kernel scaffold; baseline (speedup 1.0000x reference)
import jax
import jax.numpy as jnp
from jax.experimental import pallas as pl


def kernel(x, edge_index, Wg, bg, Wl1, bl1, Wr1, Wl2, bl2, Wr2, W1, b1, W2, b2, W3, b3):
    raise NotImplementedError("write your pallas kernel here")


# speedup vs baseline: 11.0571x; 11.0571x over previous
import functools

import jax
import jax.numpy as jnp
from jax import lax
from jax.experimental import pallas as pl
from jax.experimental.pallas import tpu as pltpu
from jax.experimental.pallas import tpu_sc as plsc

N = 10000
E = 320000
D_IN = 128
H = 16

NC = 2
NS = 16
NW = NC * NS
CHUNK = 80
NCHUNK = E // CHUNK
CPW = NCHUNK // NW
N_PAD = 10240
RPS = N_PAD // NS


def _sc_count_body(col1d, ones, zeros, out, col_v, rows_v, acc_sh, sem):
    c = lax.axis_index("c")
    s = lax.axis_index("s")
    w = c * NS + s
    pltpu.sync_copy(zeros.at[pl.ds(s * RPS, RPS)],
                    acc_sh.at[pl.ds(s * RPS, RPS)])
    pltpu.sync_copy(ones, rows_v)
    plsc.subcore_barrier()

    def step(j, carry):
        base = (w * CPW + j) * CHUNK
        pltpu.sync_copy(col1d.at[pl.ds(base, CHUNK)], col_v)
        pltpu.sync_copy(rows_v, acc_sh.at[col_v], add=True)
        return carry

    lax.fori_loop(0, CPW, step, 0)
    plsc.subcore_barrier()
    pltpu.sync_copy(acc_sh.at[pl.ds(s * RPS, RPS)],
                    out.at[c, pl.ds(s * RPS, RPS)])


def _sc_gather_add_body(row1d, col1d, src, zeros, out,
                        row_v, col_v, rows_v, acc_sh, sem):
    c = lax.axis_index("c")
    s = lax.axis_index("s")
    w = c * NS + s
    pltpu.sync_copy(zeros.at[pl.ds(s * RPS, RPS)],
                    acc_sh.at[pl.ds(s * RPS, RPS)])
    plsc.subcore_barrier()

    def step(j, carry):
        base = (w * CPW + j) * CHUNK
        pltpu.sync_copy(row1d.at[pl.ds(base, CHUNK)], row_v)
        pltpu.sync_copy(col1d.at[pl.ds(base, CHUNK)], col_v)
        pltpu.async_copy(src.at[row_v], rows_v, sem).wait()
        pltpu.sync_copy(rows_v, acc_sh.at[col_v], add=True)
        return carry

    lax.fori_loop(0, CPW, step, 0)
    plsc.subcore_barrier()
    pltpu.sync_copy(acc_sh.at[pl.ds(s * RPS, RPS)],
                    out.at[c, pl.ds(s * RPS, RPS)])


@functools.cache
def _sc_kernels():
    mesh = plsc.VectorSubcoreMesh(core_axis_name="c", subcore_axis_name="s",
                                  num_cores=NC, num_subcores=NS)
    params = pltpu.CompilerParams(use_tc_tiling_on_sc=False)
    sc_count = pl.kernel(
        _sc_count_body,
        out_type=jax.ShapeDtypeStruct((NC, N_PAD, H), jnp.float32),
        mesh=mesh,
        compiler_params=params,
        scratch_types=[
            pltpu.VMEM((CHUNK,), jnp.int32),
            pltpu.VMEM((CHUNK, H), jnp.float32),
            pltpu.VMEM_SHARED((N_PAD, H), jnp.float32),
            pltpu.SemaphoreType.DMA,
        ],
    )
    sc_gather_add = pl.kernel(
        _sc_gather_add_body,
        out_type=jax.ShapeDtypeStruct((NC, N_PAD, H), jnp.float32),
        mesh=mesh,
        compiler_params=params,
        scratch_types=[
            pltpu.VMEM((CHUNK,), jnp.int32),
            pltpu.VMEM((CHUNK,), jnp.int32),
            pltpu.VMEM((CHUNK, H), jnp.float32),
            pltpu.VMEM_SHARED((N_PAD, H), jnp.float32),
            pltpu.SemaphoreType.DMA,
        ],
    )
    return sc_count, sc_gather_add


_BLK = 2000
_GRID = N // _BLK
def _dot(a, b):
    return jnp.dot(a, b, preferred_element_type=jnp.float32)


def _tc_a_body(x_ref, wg_ref, cntp_ref, y_ref, dinv_ref, cnt_ref):
    cnt = cntp_ref[0] + cntp_ref[1]
    dinv = lax.rsqrt(cnt + 1.0)
    xw = _dot(x_ref[...], wg_ref[...])
    cnt_ref[...] = cnt
    dinv_ref[...] = dinv
    y_ref[...] = dinv * xw


def _tc_b_body(accp_ref, y_ref, dinv_ref, bg_ref, h_ref):
    acc = accp_ref[0] + accp_ref[1] + y_ref[...]
    h_ref[...] = jnp.tanh(dinv_ref[...] * acc + bg_ref[...])


def _sage_core(sp_ref, cnt_ref, h_ref, wl_ref, bl_ref, wr_ref):
    mean = (sp_ref[0] + sp_ref[1]) / jnp.maximum(cnt_ref[...], 1.0)
    o = _dot(mean, wl_ref[...]) + _dot(h_ref[...], wr_ref[...]) + bl_ref[...]
    nrm = jnp.sqrt(jnp.sum(o * o, axis=-1, keepdims=True))
    return o / jnp.maximum(nrm, 1e-12)


def _tc_sage1_body(sp_ref, cnt_ref, h_ref, wl_ref, bl_ref, wr_ref, out_ref):
    out_ref[...] = jnp.tanh(_sage_core(sp_ref, cnt_ref, h_ref, wl_ref, bl_ref, wr_ref))


def _tc_sage2_mlp_body(sp_ref, cnt_ref, h_ref, wl_ref, bl_ref, wr_ref,
                       w1_ref, b1_ref, w2_ref, b2_ref, w3_ref, b3_ref, out_ref):
    h3 = _sage_core(sp_ref, cnt_ref, h_ref, wl_ref, bl_ref, wr_ref)
    t = jax.nn.relu(_dot(h3, w1_ref[...]) + b1_ref[...])
    t = jax.nn.relu(_dot(t, w2_ref[...]) + b2_ref[...])
    out_ref[...] = _dot(t, w3_ref[...]) + b3_ref[...]


def _row_spec(width):
    return pl.BlockSpec((_BLK, width), lambda i: (i, 0))


def _part_spec():
    return pl.BlockSpec((NC, _BLK, H), lambda i: (0, i, 0))


def _full_spec(shape):
    nd = len(shape)
    return pl.BlockSpec(shape, lambda i: (0,) * nd)


def _f32(shape):
    return jax.ShapeDtypeStruct(shape, jnp.float32)


_tc_a = pl.pallas_call(
    _tc_a_body,
    grid=(_GRID,),
    in_specs=[_row_spec(D_IN), _full_spec((D_IN, H)), _part_spec()],
    out_specs=[_row_spec(H), _row_spec(H), _row_spec(H)],
    out_shape=[_f32((N, H)), _f32((N, H)), _f32((N, H))],
)

_tc_b = pl.pallas_call(
    _tc_b_body,
    grid=(_GRID,),
    in_specs=[_part_spec(), _row_spec(H), _row_spec(H), _full_spec((1, H))],
    out_specs=_row_spec(H),
    out_shape=_f32((N, H)),
)

_tc_sage1 = pl.pallas_call(
    _tc_sage1_body,
    grid=(_GRID,),
    in_specs=[_part_spec(), _row_spec(H), _row_spec(H),
              _full_spec((H, H)), _full_spec((1, H)), _full_spec((H, H))],
    out_specs=_row_spec(H),
    out_shape=_f32((N, H)),
)

_tc_sage2_mlp = pl.pallas_call(
    _tc_sage2_mlp_body,
    grid=(_GRID,),
    in_specs=[_part_spec(), _row_spec(H), _row_spec(H),
              _full_spec((H, H)), _full_spec((1, H)), _full_spec((H, H)),
              _full_spec((H, 128)), _full_spec((1, 128)),
              _full_spec((128, 128)), _full_spec((1, 128)),
              _full_spec((128, 1)), _full_spec((1, 1))],
    out_specs=_row_spec(1),
    out_shape=_f32((N, 1)),
)


def kernel(x, edge_index, Wg, bg, Wl1, bl1, Wr1, Wl2, bl2, Wr2,
           W1, b1, W2, b2, W3, b3):
    row1d = edge_index[0]
    col1d = edge_index[1]
    zeros = jnp.zeros((N_PAD, H), jnp.float32)
    ones = jnp.ones((CHUNK, H), jnp.float32)

    sc_count, sc_gather_add = _sc_kernels()
    cntp = sc_count(col1d, ones, zeros)[:, :N]
    y, dinv16, cnt16 = _tc_a(x, Wg, cntp)
    accp = sc_gather_add(row1d, col1d, y, zeros)[:, :N]
    h1 = _tc_b(accp, y, dinv16, bg.reshape(1, H))
    s2p = sc_gather_add(row1d, col1d, h1, zeros)[:, :N]
    h2 = _tc_sage1(s2p, cnt16, h1, Wl1, bl1.reshape(1, H), Wr1)
    s3p = sc_gather_add(row1d, col1d, h2, zeros)[:, :N]
    return _tc_sage2_mlp(s3p, cnt16, h2, Wl2, bl2.reshape(1, H), Wr2,
                         W1, b1.reshape(1, 128), W2, b2.reshape(1, 128),
                         W3, b3.reshape(1, 1))


# speedup vs baseline: 35.4963x; 3.2103x over previous
import functools

import jax
import jax.numpy as jnp
from jax import lax
from jax.experimental import pallas as pl
from jax.experimental.pallas import tpu as pltpu
from jax.experimental.pallas import tpu_sc as plsc

N = 10000
E = 320000
D_IN = 128
H = 16

NC = 2
NS = 16
NW = NC * NS
EPW = E // NW
CHUNK = 125
CPW = EPW // CHUNK
K = 8
N_PAD = 10240
RPS = N_PAD // NS


def _sc_count_body(col3, ones, zeros, out, col_v, rows_v, acc_sh, sem):
    c = lax.axis_index("c")
    s = lax.axis_index("s")
    w = c * NS + s
    pltpu.sync_copy(zeros.at[pl.ds(s * RPS, RPS)],
                    acc_sh.at[pl.ds(s * RPS, RPS)])
    pltpu.sync_copy(ones, rows_v)
    pltpu.sync_copy(col3.at[w], col_v)
    plsc.subcore_barrier()

    def superchunk(t, carry):
        descs = [
            pltpu.async_copy(rows_v, acc_sh.at[col_v.at[t * K + b]], sem,
                             add=True)
            for b in range(K)
        ]
        for d in descs:
            d.wait()
        return carry

    lax.fori_loop(0, CPW // K, superchunk, 0)
    plsc.subcore_barrier()
    pltpu.sync_copy(acc_sh.at[pl.ds(s * RPS, RPS)],
                    out.at[c, pl.ds(s * RPS, RPS)])


def _sc_gather_add_body(row3, col3, src, zeros, out,
                        row_v, col_v, rows_b, acc_sh, gsem, ssem):
    c = lax.axis_index("c")
    s = lax.axis_index("s")
    w = c * NS + s
    pltpu.sync_copy(zeros.at[pl.ds(s * RPS, RPS)],
                    acc_sh.at[pl.ds(s * RPS, RPS)])
    pltpu.sync_copy(row3.at[w], row_v)
    pltpu.sync_copy(col3.at[w], col_v)
    plsc.subcore_barrier()

    def superchunk(t, carry):
        base = t * K
        gd = [
            pltpu.async_copy(src.at[row_v.at[base + b]], rows_b.at[b], gsem)
            for b in range(K)
        ]
        for d in gd:
            d.wait()
        sd = [
            pltpu.async_copy(rows_b.at[b], acc_sh.at[col_v.at[base + b]],
                             ssem, add=True)
            for b in range(K)
        ]
        for d in sd:
            d.wait()
        return carry

    lax.fori_loop(0, CPW // K, superchunk, 0)
    plsc.subcore_barrier()
    pltpu.sync_copy(acc_sh.at[pl.ds(s * RPS, RPS)],
                    out.at[c, pl.ds(s * RPS, RPS)])


@functools.cache
def _sc_kernels():
    mesh = plsc.VectorSubcoreMesh(core_axis_name="c", subcore_axis_name="s",
                                  num_cores=NC, num_subcores=NS)
    params = pltpu.CompilerParams(use_tc_tiling_on_sc=False)
    sc_count = pl.kernel(
        _sc_count_body,
        out_type=jax.ShapeDtypeStruct((NC, N_PAD, H), jnp.float32),
        mesh=mesh,
        compiler_params=params,
        scratch_types=[
            pltpu.VMEM((CPW, CHUNK), jnp.int32),
            pltpu.VMEM((CHUNK, H), jnp.float32),
            pltpu.VMEM_SHARED((N_PAD, H), jnp.float32),
            pltpu.SemaphoreType.DMA,
        ],
    )
    sc_gather_add = pl.kernel(
        _sc_gather_add_body,
        out_type=jax.ShapeDtypeStruct((NC, N_PAD, H), jnp.float32),
        mesh=mesh,
        compiler_params=params,
        scratch_types=[
            pltpu.VMEM((CPW, CHUNK), jnp.int32),
            pltpu.VMEM((CPW, CHUNK), jnp.int32),
            pltpu.VMEM((K, CHUNK, H), jnp.float32),
            pltpu.VMEM_SHARED((N_PAD, H), jnp.float32),
            pltpu.SemaphoreType.DMA,
            pltpu.SemaphoreType.DMA,
        ],
    )
    return sc_count, sc_gather_add


_BLK = 2000
_GRID = N // _BLK
def _dot(a, b):
    return jnp.dot(a, b, preferred_element_type=jnp.float32)


def _tc_a_body(x_ref, wg_ref, cntp_ref, y_ref, dinv_ref, cnt_ref):
    cnt = cntp_ref[0] + cntp_ref[1]
    dinv = lax.rsqrt(cnt + 1.0)
    xw = _dot(x_ref[...], wg_ref[...])
    cnt_ref[...] = cnt
    dinv_ref[...] = dinv
    y_ref[...] = dinv * xw


def _tc_b_body(accp_ref, y_ref, dinv_ref, bg_ref, h_ref):
    acc = accp_ref[0] + accp_ref[1] + y_ref[...]
    h_ref[...] = jnp.tanh(dinv_ref[...] * acc + bg_ref[...])


def _sage_core(sp_ref, cnt_ref, h_ref, wl_ref, bl_ref, wr_ref):
    mean = (sp_ref[0] + sp_ref[1]) / jnp.maximum(cnt_ref[...], 1.0)
    o = _dot(mean, wl_ref[...]) + _dot(h_ref[...], wr_ref[...]) + bl_ref[...]
    nrm = jnp.sqrt(jnp.sum(o * o, axis=-1, keepdims=True))
    return o / jnp.maximum(nrm, 1e-12)


def _tc_sage1_body(sp_ref, cnt_ref, h_ref, wl_ref, bl_ref, wr_ref, out_ref):
    out_ref[...] = jnp.tanh(_sage_core(sp_ref, cnt_ref, h_ref, wl_ref, bl_ref, wr_ref))


def _tc_sage2_mlp_body(sp_ref, cnt_ref, h_ref, wl_ref, bl_ref, wr_ref,
                       w1_ref, b1_ref, w2_ref, b2_ref, w3_ref, b3_ref, out_ref):
    h3 = _sage_core(sp_ref, cnt_ref, h_ref, wl_ref, bl_ref, wr_ref)
    t = jax.nn.relu(_dot(h3, w1_ref[...]) + b1_ref[...])
    t = jax.nn.relu(_dot(t, w2_ref[...]) + b2_ref[...])
    out_ref[...] = _dot(t, w3_ref[...]) + b3_ref[...]


def _row_spec(width):
    return pl.BlockSpec((_BLK, width), lambda i: (i, 0))


def _part_spec():
    return pl.BlockSpec((NC, _BLK, H), lambda i: (0, i, 0))


def _full_spec(shape):
    nd = len(shape)
    return pl.BlockSpec(shape, lambda i: (0,) * nd)


def _f32(shape):
    return jax.ShapeDtypeStruct(shape, jnp.float32)


_tc_a = pl.pallas_call(
    _tc_a_body,
    grid=(_GRID,),
    in_specs=[_row_spec(D_IN), _full_spec((D_IN, H)), _part_spec()],
    out_specs=[_row_spec(H), _row_spec(H), _row_spec(H)],
    out_shape=[_f32((N, H)), _f32((N, H)), _f32((N, H))],
)

_tc_b = pl.pallas_call(
    _tc_b_body,
    grid=(_GRID,),
    in_specs=[_part_spec(), _row_spec(H), _row_spec(H), _full_spec((1, H))],
    out_specs=_row_spec(H),
    out_shape=_f32((N, H)),
)

_tc_sage1 = pl.pallas_call(
    _tc_sage1_body,
    grid=(_GRID,),
    in_specs=[_part_spec(), _row_spec(H), _row_spec(H),
              _full_spec((H, H)), _full_spec((1, H)), _full_spec((H, H))],
    out_specs=_row_spec(H),
    out_shape=_f32((N, H)),
)

_tc_sage2_mlp = pl.pallas_call(
    _tc_sage2_mlp_body,
    grid=(_GRID,),
    in_specs=[_part_spec(), _row_spec(H), _row_spec(H),
              _full_spec((H, H)), _full_spec((1, H)), _full_spec((H, H)),
              _full_spec((H, 128)), _full_spec((1, 128)),
              _full_spec((128, 128)), _full_spec((1, 128)),
              _full_spec((128, 1)), _full_spec((1, 1))],
    out_specs=_row_spec(1),
    out_shape=_f32((N, 1)),
)


def kernel(x, edge_index, Wg, bg, Wl1, bl1, Wr1, Wl2, bl2, Wr2,
           W1, b1, W2, b2, W3, b3):
    row3 = edge_index[0].reshape(NW, CPW, CHUNK)
    col3 = edge_index[1].reshape(NW, CPW, CHUNK)
    zeros = jnp.zeros((N_PAD, H), jnp.float32)
    ones = jnp.ones((CHUNK, H), jnp.float32)

    sc_count, sc_gather_add = _sc_kernels()
    cntp = sc_count(col3, ones, zeros)[:, :N]
    y, dinv16, cnt16 = _tc_a(x, Wg, cntp)
    accp = sc_gather_add(row3, col3, y, zeros)[:, :N]
    h1 = _tc_b(accp, y, dinv16, bg.reshape(1, H))
    s2p = sc_gather_add(row3, col3, h1, zeros)[:, :N]
    h2 = _tc_sage1(s2p, cnt16, h1, Wl1, bl1.reshape(1, H), Wr1)
    s3p = sc_gather_add(row3, col3, h2, zeros)[:, :N]
    return _tc_sage2_mlp(s3p, cnt16, h2, Wl2, bl2.reshape(1, H), Wr2,
                         W1, b1.reshape(1, 128), W2, b2.reshape(1, 128),
                         W3, b3.reshape(1, 1))


# speedup vs baseline: 52.1795x; 1.4700x over previous
import functools

import jax
import jax.numpy as jnp
from jax import lax
from jax.experimental import pallas as pl
from jax.experimental.pallas import tpu as pltpu
from jax.experimental.pallas import tpu_sc as plsc

N = 10000
E = 320000
D_IN = 128
H = 16

NC = 2
NS = 16
NW = NC * NS
EPW = E // NW
CHUNK = 125
CPW = EPW // CHUNK
K = 8
N_PAD = 10240
RPS = N_PAD // NS
TAIL = N - (NS - 1) * RPS
PR = N // 8


def _sc_count_body(col3, ones, zeros, out, col_v, rows_v, acc_sh, sem):
    c = lax.axis_index("c")
    s = lax.axis_index("s")
    w = c * NS + s
    pltpu.sync_copy(zeros.at[pl.ds(s * RPS, RPS)],
                    acc_sh.at[pl.ds(s * RPS, RPS)])
    pltpu.sync_copy(ones, rows_v)
    pltpu.sync_copy(col3.at[w], col_v)
    plsc.subcore_barrier()

    def superchunk(t, carry):
        descs = [
            pltpu.async_copy(rows_v, acc_sh.at[col_v.at[t * K + b]], sem,
                             add=True)
            for b in range(K)
        ]
        for d in descs:
            d.wait()
        return carry

    lax.fori_loop(0, CPW // K, superchunk, 0)
    plsc.subcore_barrier()
    _copy_out(c, s, acc_sh, out)


def _copy_out(c, s, acc_sh, out):
    @pl.when(s < NS - 1)
    def _():
        pltpu.sync_copy(acc_sh.at[pl.ds(s * RPS, RPS)],
                        out.at[c, pl.ds(s * RPS, RPS)])

    @pl.when(s == NS - 1)
    def _():
        pltpu.sync_copy(acc_sh.at[pl.ds((NS - 1) * RPS, TAIL)],
                        out.at[c, pl.ds((NS - 1) * RPS, TAIL)])


def _sc_gather_add_body(row3, col3, src, zeros, out,
                        row_v, col_v, rows_b, acc_sh, gsem, ssem):
    c = lax.axis_index("c")
    s = lax.axis_index("s")
    w = c * NS + s
    pltpu.sync_copy(zeros.at[pl.ds(s * RPS, RPS)],
                    acc_sh.at[pl.ds(s * RPS, RPS)])
    pltpu.sync_copy(row3.at[w], row_v)
    pltpu.sync_copy(col3.at[w], col_v)
    plsc.subcore_barrier()

    def superchunk(t, carry):
        base = t * K
        gd = [
            pltpu.async_copy(src.at[row_v.at[base + b]], rows_b.at[b], gsem)
            for b in range(K)
        ]
        for d in gd:
            d.wait()
        sd = [
            pltpu.async_copy(rows_b.at[b], acc_sh.at[col_v.at[base + b]],
                             ssem, add=True)
            for b in range(K)
        ]
        for d in sd:
            d.wait()
        return carry

    lax.fori_loop(0, CPW // K, superchunk, 0)
    plsc.subcore_barrier()
    _copy_out(c, s, acc_sh, out)


@functools.cache
def _sc_kernels():
    mesh = plsc.VectorSubcoreMesh(core_axis_name="c", subcore_axis_name="s",
                                  num_cores=NC, num_subcores=NS)
    params = pltpu.CompilerParams(use_tc_tiling_on_sc=False)
    sc_count = pl.kernel(
        _sc_count_body,
        out_type=jax.ShapeDtypeStruct((NC, N, H), jnp.float32),
        mesh=mesh,
        compiler_params=params,
        scratch_types=[
            pltpu.VMEM((CPW, CHUNK), jnp.int32),
            pltpu.VMEM((CHUNK, H), jnp.float32),
            pltpu.VMEM_SHARED((N_PAD, H), jnp.float32),
            pltpu.SemaphoreType.DMA,
        ],
    )
    sc_gather_add = pl.kernel(
        _sc_gather_add_body,
        out_type=jax.ShapeDtypeStruct((NC, N, H), jnp.float32),
        mesh=mesh,
        compiler_params=params,
        scratch_types=[
            pltpu.VMEM((CPW, CHUNK), jnp.int32),
            pltpu.VMEM((CPW, CHUNK), jnp.int32),
            pltpu.VMEM((K, CHUNK, H), jnp.float32),
            pltpu.VMEM_SHARED((N_PAD, H), jnp.float32),
            pltpu.SemaphoreType.DMA,
            pltpu.SemaphoreType.DMA,
        ],
    )
    return sc_count, sc_gather_add


_HIGH = jax.lax.Precision.HIGHEST


def _dot(a, b):
    return jnp.dot(a, b, preferred_element_type=jnp.float32)


def _tc_xw_body(x8_ref, wg8_ref, xwp_ref):
    xwp_ref[...] = _dot(x8_ref[...], wg8_ref[...])


def _tc_y_body(cntp_ref, xwp_ref, yp_ref, dinvp_ref, cntq_ref):
    cnt = cntp_ref[0] + cntp_ref[1]
    dinv = lax.rsqrt(cnt + 1.0)
    cntq_ref[...] = cnt
    dinvp_ref[...] = dinv
    yp_ref[...] = dinv * xwp_ref[...]


def _tc_b_body(accp_ref, yp_ref, dinvp_ref, bgt_ref, hp_ref):
    acc = accp_ref[0] + accp_ref[1] + yp_ref[...]
    hp_ref[...] = jnp.tanh(dinvp_ref[...] * acc + bgt_ref[...])


def _sage_core(sp_ref, cntq_ref, hp_ref, wl8_ref, blt_ref, wr8_ref, gs_ref):
    mean_p = (sp_ref[0] + sp_ref[1]) / jnp.maximum(cntq_ref[...], 1.0)
    o = _dot(mean_p, wl8_ref[...]) + _dot(hp_ref[...], wr8_ref[...]) + blt_ref[...]
    nrm2 = jnp.dot(o * o, gs_ref[...], precision=_HIGH,
                   preferred_element_type=jnp.float32)
    nrm = jnp.sqrt(nrm2)
    return o / jnp.maximum(nrm, 1e-12)


def _tc_sage1_body(sp_ref, cntq_ref, hp_ref, wl8_ref, blt_ref, wr8_ref,
                   gs_ref, out_ref):
    out_ref[...] = jnp.tanh(
        _sage_core(sp_ref, cntq_ref, hp_ref, wl8_ref, blt_ref, wr8_ref, gs_ref))


def _tc_sage2_mlp_body(sp_ref, cntq_ref, hp_ref, wl8_ref, blt_ref, wr8_ref,
                       gs_ref, w18_ref, b1t_ref, w2_ref, b2_ref, w3_ref,
                       b3_ref, out_ref):
    h3 = _sage_core(sp_ref, cntq_ref, hp_ref, wl8_ref, blt_ref, wr8_ref, gs_ref)
    t1 = jax.nn.relu(_dot(h3, w18_ref[...]) + b1t_ref[...])
    cols = []
    for a in range(8):
        t2 = jax.nn.relu(_dot(t1[:, 128 * a:128 * (a + 1)], w2_ref[...])
                         + b2_ref[...])
        cols.append(_dot(t2, w3_ref[...]) + b3_ref[...])
    out_ref[...] = jnp.concatenate(cols, axis=1)


def _full_spec(shape):
    nd = len(shape)
    return pl.BlockSpec(shape, lambda: (0,) * nd)


def _f32(shape):
    return jax.ShapeDtypeStruct(shape, jnp.float32)


_tc_xw = pl.pallas_call(
    _tc_xw_body,
    out_shape=_f32((PR, 128)),
)

_tc_y = pl.pallas_call(
    _tc_y_body,
    out_shape=[_f32((PR, 128)), _f32((PR, 128)), _f32((PR, 128))],
)

_tc_b = pl.pallas_call(
    _tc_b_body,
    out_shape=_f32((PR, 128)),
)

_tc_sage1 = pl.pallas_call(
    _tc_sage1_body,
    out_shape=_f32((PR, 128)),
)

_tc_sage2_mlp = pl.pallas_call(
    _tc_sage2_mlp_body,
    out_shape=_f32((PR, 8)),
)


def kernel(x, edge_index, Wg, bg, Wl1, bl1, Wr1, Wl2, bl2, Wr2,
           W1, b1, W2, b2, W3, b3):
    row3 = edge_index[0].reshape(NW, CPW, CHUNK)
    col3 = edge_index[1].reshape(NW, CPW, CHUNK)
    zeros = jnp.zeros((N_PAD, H), jnp.float32)
    ones = jnp.ones((CHUNK, H), jnp.float32)

    eye8 = jnp.eye(8, dtype=jnp.float32)
    x8 = x.reshape(PR, 8 * D_IN)
    wg8 = jnp.kron(eye8, Wg)
    wl18 = jnp.kron(eye8, Wl1)
    wr18 = jnp.kron(eye8, Wr1)
    wl28 = jnp.kron(eye8, Wl2)
    wr28 = jnp.kron(eye8, Wr2)
    w18 = jnp.kron(eye8, W1)
    gs = jnp.kron(eye8, jnp.ones((H, H), jnp.float32))
    bgt = jnp.tile(bg, 8).reshape(1, 128)
    bl1t = jnp.tile(bl1, 8).reshape(1, 128)
    bl2t = jnp.tile(bl2, 8).reshape(1, 128)
    b1t = jnp.tile(b1, 8).reshape(1, 8 * 128)

    sc_count, sc_gather_add = _sc_kernels()
    cntp = sc_count(col3, ones, zeros).reshape(NC, PR, 128)
    xw_p = _tc_xw(x8, wg8)
    y_p, dinv_p, cnt_p = _tc_y(cntp, xw_p)
    accp = sc_gather_add(row3, col3, y_p.reshape(N, H), zeros)
    h1_p = _tc_b(accp.reshape(NC, PR, 128), y_p, dinv_p, bgt)
    s2p = sc_gather_add(row3, col3, h1_p.reshape(N, H), zeros)
    h2_p = _tc_sage1(s2p.reshape(NC, PR, 128), cnt_p, h1_p,
                     wl18, bl1t, wr18, gs)
    s3p = sc_gather_add(row3, col3, h2_p.reshape(N, H), zeros)
    out_p = _tc_sage2_mlp(s3p.reshape(NC, PR, 128), cnt_p, h2_p,
                          wl28, bl2t, wr28, gs,
                          w18, b1t, W2, b2.reshape(1, 128),
                          W3, b3.reshape(1, 1))
    return out_p.reshape(N, 1)


# speedup vs baseline: 61.3918x; 1.1766x over previous
import functools

import jax
import jax.numpy as jnp
from jax import lax
from jax.experimental import pallas as pl
from jax.experimental.pallas import tpu as pltpu
from jax.experimental.pallas import tpu_sc as plsc

N = 10000
E = 320000
D_IN = 128
H = 16

NC = 2
NS = 16
NW = NC * NS
EPW = E // NW
CHUNK = 125
CPW = EPW // CHUNK
K = 8
N_PAD = 10240
RPS = N_PAD // NS
TAIL = N - (NS - 1) * RPS
PR = N // 8


T = CPW // K


def _sc_count_body(edges, ones, zeros, out, col_v, rows_v, acc_sh, sem):
    c = lax.axis_index("c")
    s = lax.axis_index("s")
    w = c * NS + s
    pltpu.sync_copy(zeros.at[pl.ds(s * RPS, RPS)],
                    acc_sh.at[pl.ds(s * RPS, RPS)])
    pltpu.sync_copy(ones, rows_v)
    pltpu.sync_copy(edges.at[1, w], col_v)
    plsc.subcore_barrier()

    def superchunk(t, carry):
        descs = [
            pltpu.async_copy(rows_v, acc_sh.at[col_v.at[t * K + b]], sem,
                             add=True)
            for b in range(K)
        ]
        for d in descs:
            d.wait()
        return carry

    lax.fori_loop(0, CPW // K, superchunk, 0)
    plsc.subcore_barrier()
    _copy_out(c, s, acc_sh, out)


def _copy_out(c, s, acc_sh, out):
    @pl.when(s < NS - 1)
    def _():
        pltpu.sync_copy(acc_sh.at[pl.ds(s * RPS, RPS)],
                        out.at[c, pl.ds(s * RPS, RPS)])

    @pl.when(s == NS - 1)
    def _():
        pltpu.sync_copy(acc_sh.at[pl.ds((NS - 1) * RPS, TAIL)],
                        out.at[c, pl.ds((NS - 1) * RPS, TAIL)])


def _sc_gather_add_body(edges, src, zeros, out,
                        row_v, col_v, rows_b, acc_sh, gsem, ssem):
    c = lax.axis_index("c")
    s = lax.axis_index("s")
    w = c * NS + s
    pltpu.sync_copy(zeros.at[pl.ds(s * RPS, RPS)],
                    acc_sh.at[pl.ds(s * RPS, RPS)])
    pltpu.sync_copy(edges.at[0, w], row_v)
    pltpu.sync_copy(edges.at[1, w], col_v)
    plsc.subcore_barrier()

    def fire_g(t, set_):
        for b in range(K):
            pltpu.async_copy(src.at[row_v.at[t * K + b]],
                             rows_b.at[set_, b], gsem)

    def wait_g(t, set_):
        for b in range(K):
            pltpu.make_async_copy(src.at[row_v.at[t * K + b]],
                                  rows_b.at[set_, b], gsem).wait()

    def fire_s(t, set_):
        for b in range(K):
            pltpu.async_copy(rows_b.at[set_, b],
                             acc_sh.at[col_v.at[t * K + b]], ssem, add=True)

    def wait_s(t, set_):
        for b in range(K):
            pltpu.make_async_copy(rows_b.at[set_, b],
                                  acc_sh.at[col_v.at[t * K + b]], ssem).wait()

    fire_g(0, 0)

    def body(t, carry):
        p = lax.rem(t, 2)
        wait_g(t, p)

        @pl.when(t > 0)
        def _():
            wait_s(t - 1, 1 - p)

        @pl.when(t < T - 1)
        def _():
            fire_g(t + 1, 1 - p)

        fire_s(t, p)
        return carry

    lax.fori_loop(0, T, body, 0)
    wait_s(T - 1, (T - 1) % 2)
    plsc.subcore_barrier()
    _copy_out(c, s, acc_sh, out)


@functools.cache
def _sc_kernels():
    mesh = plsc.VectorSubcoreMesh(core_axis_name="c", subcore_axis_name="s",
                                  num_cores=NC, num_subcores=NS)
    params = pltpu.CompilerParams(use_tc_tiling_on_sc=False)
    sc_count = pl.kernel(
        _sc_count_body,
        out_type=jax.ShapeDtypeStruct((NC, N, H), jnp.float32),
        mesh=mesh,
        compiler_params=params,
        scratch_types=[
            pltpu.VMEM((CPW, CHUNK), jnp.int32),
            pltpu.VMEM((CHUNK, H), jnp.float32),
            pltpu.VMEM_SHARED((N_PAD, H), jnp.float32),
            pltpu.SemaphoreType.DMA,
        ],
    )
    sc_gather_add = pl.kernel(
        _sc_gather_add_body,
        out_type=jax.ShapeDtypeStruct((NC, N, H), jnp.float32),
        mesh=mesh,
        compiler_params=params,
        scratch_types=[
            pltpu.VMEM((CPW, CHUNK), jnp.int32),
            pltpu.VMEM((CPW, CHUNK), jnp.int32),
            pltpu.VMEM((2, K, CHUNK, H), jnp.float32),
            pltpu.VMEM_SHARED((N_PAD, H), jnp.float32),
            pltpu.SemaphoreType.DMA,
            pltpu.SemaphoreType.DMA,
        ],
    )
    return sc_count, sc_gather_add


_HIGH = jax.lax.Precision.HIGHEST


def _dot(a, b):
    return jnp.dot(a, b, preferred_element_type=jnp.float32)


def _tc_xw_body(x8_ref, wg8_ref, xwp_ref):
    xwp_ref[...] = _dot(x8_ref[...], wg8_ref[...])


def _tc_y_body(cntp_ref, xwp_ref, yp_ref, dinvp_ref, cntq_ref):
    cnt = cntp_ref[0] + cntp_ref[1]
    dinv = lax.rsqrt(cnt + 1.0)
    cntq_ref[...] = cnt
    dinvp_ref[...] = dinv
    yp_ref[...] = dinv * xwp_ref[...]


def _tc_b_body(accp_ref, yp_ref, dinvp_ref, bgt_ref, hp_ref):
    acc = accp_ref[0] + accp_ref[1] + yp_ref[...]
    hp_ref[...] = jnp.tanh(dinvp_ref[...] * acc + bgt_ref[...])


def _sage_core(sp_ref, cntq_ref, hp_ref, wl8_ref, blt_ref, wr8_ref, gs_ref):
    mean_p = (sp_ref[0] + sp_ref[1]) / jnp.maximum(cntq_ref[...], 1.0)
    o = _dot(mean_p, wl8_ref[...]) + _dot(hp_ref[...], wr8_ref[...]) + blt_ref[...]
    nrm2 = jnp.dot(o * o, gs_ref[...], precision=_HIGH,
                   preferred_element_type=jnp.float32)
    nrm = jnp.sqrt(nrm2)
    return o / jnp.maximum(nrm, 1e-12)


def _tc_sage1_body(sp_ref, cntq_ref, hp_ref, wl8_ref, blt_ref, wr8_ref,
                   gs_ref, out_ref):
    out_ref[...] = jnp.tanh(
        _sage_core(sp_ref, cntq_ref, hp_ref, wl8_ref, blt_ref, wr8_ref, gs_ref))


def _tc_sage2_mlp_body(sp_ref, cntq_ref, hp_ref, wl8_ref, blt_ref, wr8_ref,
                       gs_ref, w18_ref, b1t_ref, w2_ref, b2_ref, w3_ref,
                       b3_ref, out_ref):
    h3 = _sage_core(sp_ref, cntq_ref, hp_ref, wl8_ref, blt_ref, wr8_ref, gs_ref)
    t1 = jax.nn.relu(_dot(h3, w18_ref[...]) + b1t_ref[...])
    cols = []
    for a in range(8):
        t2 = jax.nn.relu(_dot(t1[:, 128 * a:128 * (a + 1)], w2_ref[...])
                         + b2_ref[...])
        cols.append(_dot(t2, w3_ref[...]) + b3_ref[...])
    out_ref[...] = jnp.concatenate(cols, axis=1)


def _full_spec(shape):
    nd = len(shape)
    return pl.BlockSpec(shape, lambda: (0,) * nd)


def _f32(shape):
    return jax.ShapeDtypeStruct(shape, jnp.float32)


_tc_xw = pl.pallas_call(
    _tc_xw_body,
    out_shape=_f32((PR, 128)),
)

_tc_y = pl.pallas_call(
    _tc_y_body,
    out_shape=[_f32((PR, 128)), _f32((PR, 128)), _f32((PR, 128))],
)

_tc_b = pl.pallas_call(
    _tc_b_body,
    out_shape=_f32((PR, 128)),
)

_tc_sage1 = pl.pallas_call(
    _tc_sage1_body,
    out_shape=_f32((PR, 128)),
)

_tc_sage2_mlp = pl.pallas_call(
    _tc_sage2_mlp_body,
    out_shape=_f32((PR, 8)),
)


def kernel(x, edge_index, Wg, bg, Wl1, bl1, Wr1, Wl2, bl2, Wr2,
           W1, b1, W2, b2, W3, b3):
    edges = edge_index.reshape(2, NW, CPW, CHUNK)
    zeros = jnp.zeros((N_PAD, H), jnp.float32)
    ones = jnp.ones((CHUNK, H), jnp.float32)

    eye8 = jnp.eye(8, dtype=jnp.float32)
    x8 = x.reshape(PR, 8 * D_IN)
    wg8 = jnp.kron(eye8, Wg)
    wl18 = jnp.kron(eye8, Wl1)
    wr18 = jnp.kron(eye8, Wr1)
    wl28 = jnp.kron(eye8, Wl2)
    wr28 = jnp.kron(eye8, Wr2)
    w18 = jnp.kron(eye8, W1)
    gs = jnp.kron(eye8, jnp.ones((H, H), jnp.float32))
    bgt = jnp.tile(bg, 8).reshape(1, 128)
    bl1t = jnp.tile(bl1, 8).reshape(1, 128)
    bl2t = jnp.tile(bl2, 8).reshape(1, 128)
    b1t = jnp.tile(b1, 8).reshape(1, 8 * 128)

    sc_count, sc_gather_add = _sc_kernels()
    cntp = sc_count(edges, ones, zeros).reshape(NC, PR, 128)
    xw_p = _tc_xw(x8, wg8)
    y_p, dinv_p, cnt_p = _tc_y(cntp, xw_p)
    accp = sc_gather_add(edges, y_p.reshape(N, H), zeros)
    h1_p = _tc_b(accp.reshape(NC, PR, 128), y_p, dinv_p, bgt)
    s2p = sc_gather_add(edges, h1_p.reshape(N, H), zeros)
    h2_p = _tc_sage1(s2p.reshape(NC, PR, 128), cnt_p, h1_p,
                     wl18, bl1t, wr18, gs)
    s3p = sc_gather_add(edges, h2_p.reshape(N, H), zeros)
    out_p = _tc_sage2_mlp(s3p.reshape(NC, PR, 128), cnt_p, h2_p,
                          wl28, bl2t, wr28, gs,
                          w18, b1t, W2, b2.reshape(1, 128),
                          W3, b3.reshape(1, 1))
    return out_p.reshape(N, 1)


# speedup vs baseline: 64.9258x; 1.0576x over previous
import functools

import jax
import jax.numpy as jnp
from jax import lax
from jax.experimental import pallas as pl
from jax.experimental.pallas import tpu as pltpu
from jax.experimental.pallas import tpu_sc as plsc

N = 10000
E = 320000
D_IN = 128
H = 16

NC = 2
NS = 16
NW = NC * NS
EPW = E // NW
CHUNK = 125
CPW = EPW // CHUNK
K = 10
N_PAD = 10240
RPS = N_PAD // NS
TAIL = N - (NS - 1) * RPS
PR = N // 8


T = CPW // K


def _sc_count_body(edges, ones, zeros, out, col_v, rows_v, acc_sh, sem):
    c = lax.axis_index("c")
    s = lax.axis_index("s")
    w = c * NS + s
    pltpu.sync_copy(zeros.at[pl.ds(s * RPS, RPS)],
                    acc_sh.at[pl.ds(s * RPS, RPS)])
    pltpu.sync_copy(ones, rows_v)
    pltpu.sync_copy(edges.at[1, w], col_v)
    plsc.subcore_barrier()

    def superchunk(t, carry):
        descs = [
            pltpu.async_copy(rows_v, acc_sh.at[col_v.at[t * K + b]], sem,
                             add=True)
            for b in range(K)
        ]
        for d in descs:
            d.wait()
        return carry

    lax.fori_loop(0, CPW // K, superchunk, 0)
    plsc.subcore_barrier()
    _copy_out(c, s, acc_sh, out)


def _copy_out(c, s, acc_sh, out):
    @pl.when(s < NS - 1)
    def _():
        pltpu.sync_copy(acc_sh.at[pl.ds(s * RPS, RPS)],
                        out.at[c, pl.ds(s * RPS, RPS)])

    @pl.when(s == NS - 1)
    def _():
        pltpu.sync_copy(acc_sh.at[pl.ds((NS - 1) * RPS, TAIL)],
                        out.at[c, pl.ds((NS - 1) * RPS, TAIL)])


def _sc_gather_add_body(edges, src, zeros, out,
                        row_v, col_v, rows_b, acc_sh, gsem, ssem):
    c = lax.axis_index("c")
    s = lax.axis_index("s")
    w = c * NS + s
    d0 = pltpu.async_copy(zeros.at[pl.ds(s * RPS, RPS)],
                          acc_sh.at[pl.ds(s * RPS, RPS)], gsem)
    d1 = pltpu.async_copy(edges.at[0, w], row_v, gsem)
    d2 = pltpu.async_copy(edges.at[1, w], col_v, gsem)
    d0.wait()
    d1.wait()
    d2.wait()
    plsc.subcore_barrier()

    def fire_g(t, set_):
        for b in range(K):
            pltpu.async_copy(src.at[row_v.at[t * K + b]],
                             rows_b.at[set_, b], gsem)

    def wait_g(t, set_):
        for b in range(K):
            pltpu.make_async_copy(src.at[row_v.at[t * K + b]],
                                  rows_b.at[set_, b], gsem).wait()

    def fire_s(t, set_):
        for b in range(K):
            pltpu.async_copy(rows_b.at[set_, b],
                             acc_sh.at[col_v.at[t * K + b]], ssem, add=True)

    def wait_s(t, set_):
        for b in range(K):
            pltpu.make_async_copy(rows_b.at[set_, b],
                                  acc_sh.at[col_v.at[t * K + b]], ssem).wait()

    fire_g(0, 0)

    def body(t, carry):
        p = lax.rem(t, 2)
        wait_g(t, p)

        @pl.when(t > 0)
        def _():
            wait_s(t - 1, 1 - p)

        @pl.when(t < T - 1)
        def _():
            fire_g(t + 1, 1 - p)

        fire_s(t, p)
        return carry

    lax.fori_loop(0, T, body, 0)
    wait_s(T - 1, (T - 1) % 2)
    plsc.subcore_barrier()
    _copy_out(c, s, acc_sh, out)


@functools.cache
def _sc_kernels():
    mesh = plsc.VectorSubcoreMesh(core_axis_name="c", subcore_axis_name="s",
                                  num_cores=NC, num_subcores=NS)
    params = pltpu.CompilerParams(use_tc_tiling_on_sc=False)
    sc_count = pl.kernel(
        _sc_count_body,
        out_type=jax.ShapeDtypeStruct((NC, N, H), jnp.float32),
        mesh=mesh,
        compiler_params=params,
        scratch_types=[
            pltpu.VMEM((CPW, CHUNK), jnp.int32),
            pltpu.VMEM((CHUNK, H), jnp.float32),
            pltpu.VMEM_SHARED((N_PAD, H), jnp.float32),
            pltpu.SemaphoreType.DMA,
        ],
    )
    sc_gather_add = pl.kernel(
        _sc_gather_add_body,
        out_type=jax.ShapeDtypeStruct((NC, N, H), jnp.float32),
        mesh=mesh,
        compiler_params=params,
        scratch_types=[
            pltpu.VMEM((CPW, CHUNK), jnp.int32),
            pltpu.VMEM((CPW, CHUNK), jnp.int32),
            pltpu.VMEM((2, K, CHUNK, H), jnp.float32),
            pltpu.VMEM_SHARED((N_PAD, H), jnp.float32),
            pltpu.SemaphoreType.DMA,
            pltpu.SemaphoreType.DMA,
        ],
    )
    return sc_count, sc_gather_add


_HIGH = jax.lax.Precision.HIGHEST


def _dot(a, b):
    return jnp.dot(a, b, preferred_element_type=jnp.float32)


def _tc_xw_body(x8_ref, wg8_ref, xwp_ref):
    xwp_ref[...] = _dot(x8_ref[...], wg8_ref[...])


def _tc_y_body(cntp_ref, xwp_ref, yp_ref, dinvp_ref, cntq_ref):
    cnt = cntp_ref[0] + cntp_ref[1]
    dinv = lax.rsqrt(cnt + 1.0)
    cntq_ref[...] = cnt
    dinvp_ref[...] = dinv
    yp_ref[...] = dinv * xwp_ref[...]


def _tc_b_body(accp_ref, yp_ref, dinvp_ref, bgt_ref, hp_ref):
    acc = accp_ref[0] + accp_ref[1] + yp_ref[...]
    hp_ref[...] = jnp.tanh(dinvp_ref[...] * acc + bgt_ref[...])


def _sage_core(sp_ref, cntq_ref, hp_ref, wl8_ref, blt_ref, wr8_ref, gs_ref):
    mean_p = (sp_ref[0] + sp_ref[1]) / jnp.maximum(cntq_ref[...], 1.0)
    o = _dot(mean_p, wl8_ref[...]) + _dot(hp_ref[...], wr8_ref[...]) + blt_ref[...]
    nrm2 = jnp.dot(o * o, gs_ref[...], precision=_HIGH,
                   preferred_element_type=jnp.float32)
    nrm = jnp.sqrt(nrm2)
    return o / jnp.maximum(nrm, 1e-12)


def _tc_sage1_body(sp_ref, cntq_ref, hp_ref, wl8_ref, blt_ref, wr8_ref,
                   gs_ref, out_ref):
    out_ref[...] = jnp.tanh(
        _sage_core(sp_ref, cntq_ref, hp_ref, wl8_ref, blt_ref, wr8_ref, gs_ref))


def _tc_sage2_mlp_body(sp_ref, cntq_ref, hp_ref, wl8_ref, blt_ref, wr8_ref,
                       gs_ref, w18_ref, b1t_ref, w2_ref, b2_ref, w3_ref,
                       b3_ref, out_ref):
    h3 = _sage_core(sp_ref, cntq_ref, hp_ref, wl8_ref, blt_ref, wr8_ref, gs_ref)
    t1 = jax.nn.relu(_dot(h3, w18_ref[...]) + b1t_ref[...])
    cols = []
    for a in range(8):
        t2 = jax.nn.relu(_dot(t1[:, 128 * a:128 * (a + 1)], w2_ref[...])
                         + b2_ref[...])
        cols.append(_dot(t2, w3_ref[...]) + b3_ref[...])
    out_ref[...] = jnp.concatenate(cols, axis=1)


def _full_spec(shape):
    nd = len(shape)
    return pl.BlockSpec(shape, lambda: (0,) * nd)


def _f32(shape):
    return jax.ShapeDtypeStruct(shape, jnp.float32)


_tc_xw = pl.pallas_call(
    _tc_xw_body,
    out_shape=_f32((PR, 128)),
)

_tc_y = pl.pallas_call(
    _tc_y_body,
    out_shape=[_f32((PR, 128)), _f32((PR, 128)), _f32((PR, 128))],
)

_tc_b = pl.pallas_call(
    _tc_b_body,
    out_shape=_f32((PR, 128)),
)

_tc_sage1 = pl.pallas_call(
    _tc_sage1_body,
    out_shape=_f32((PR, 128)),
)

_tc_sage2_mlp = pl.pallas_call(
    _tc_sage2_mlp_body,
    out_shape=_f32((PR, 8)),
)


def kernel(x, edge_index, Wg, bg, Wl1, bl1, Wr1, Wl2, bl2, Wr2,
           W1, b1, W2, b2, W3, b3):
    edges = edge_index.reshape(2, NW, CPW, CHUNK)
    zeros = jnp.zeros((N_PAD, H), jnp.float32)
    ones = jnp.ones((CHUNK, H), jnp.float32)

    eye8 = jnp.eye(8, dtype=jnp.float32)
    x8 = x.reshape(PR, 8 * D_IN)
    wg8 = jnp.kron(eye8, Wg)
    wl18 = jnp.kron(eye8, Wl1)
    wr18 = jnp.kron(eye8, Wr1)
    wl28 = jnp.kron(eye8, Wl2)
    wr28 = jnp.kron(eye8, Wr2)
    w18 = jnp.kron(eye8, W1)
    gs = jnp.kron(eye8, jnp.ones((H, H), jnp.float32))
    bgt = jnp.tile(bg, 8).reshape(1, 128)
    bl1t = jnp.tile(bl1, 8).reshape(1, 128)
    bl2t = jnp.tile(bl2, 8).reshape(1, 128)
    b1t = jnp.tile(b1, 8).reshape(1, 8 * 128)

    sc_count, sc_gather_add = _sc_kernels()
    cntp = sc_count(edges, ones, zeros).reshape(NC, PR, 128)
    xw_p = _tc_xw(x8, wg8)
    y_p, dinv_p, cnt_p = _tc_y(cntp, xw_p)
    accp = sc_gather_add(edges, y_p.reshape(N, H), zeros)
    h1_p = _tc_b(accp.reshape(NC, PR, 128), y_p, dinv_p, bgt)
    s2p = sc_gather_add(edges, h1_p.reshape(N, H), zeros)
    h2_p = _tc_sage1(s2p.reshape(NC, PR, 128), cnt_p, h1_p,
                     wl18, bl1t, wr18, gs)
    s3p = sc_gather_add(edges, h2_p.reshape(N, H), zeros)
    out_p = _tc_sage2_mlp(s3p.reshape(NC, PR, 128), cnt_p, h2_p,
                          wl28, bl2t, wr28, gs,
                          w18, b1t, W2, b2.reshape(1, 128),
                          W3, b3.reshape(1, 1))
    return out_p.reshape(N, 1)
